# Initial kernel scaffold; baseline (speedup 1.0000x reference)
import jax
import jax.numpy as jnp
from jax.experimental import pallas as pl


def kernel(x, text_table, pos_table):
    raise NotImplementedError("write your pallas kernel here")


# speedup vs baseline: 3.6835x; 3.6835x over previous
import functools

import jax
import jax.numpy as jnp
from jax import lax
from jax.experimental import pallas as pl
from jax.experimental.pallas import tpu as pltpu
from jax.experimental.pallas import tpu_sc as plsc

NC = 2
NS = 16
NW = NC * NS

CH = 4


def _build(B, S, V, D):
    rows = B * S
    assert rows % NW == 0
    rows_per_w = rows // NW
    ch_rows = CH * S
    assert rows_per_w % ch_rows == 0
    nchunk = rows_per_w // ch_rows

    mesh = plsc.VectorSubcoreMesh(core_axis_name="c", subcore_axis_name="s")

    @functools.partial(
        pl.kernel,
        mesh=mesh,
        compiler_params=pltpu.CompilerParams(use_tc_tiling_on_sc=False),
        out_type=jax.ShapeDtypeStruct((rows, D), jnp.float32),
        scratch_types=[
            pltpu.VMEM((ch_rows,), jnp.int32),
            pltpu.VMEM((ch_rows, D), jnp.float32),
            pltpu.VMEM((S, D), jnp.float32),
        ],
    )
    def emb(x_hbm, table_hbm, pos_hbm, out_hbm, idx_v, rows_v, pos_v):
        cid = lax.axis_index("c")
        sid = lax.axis_index("s")
        wid = sid * NC + cid

        pltpu.sync_copy(pos_hbm, pos_v)

        base = wid * rows_per_w
        for c in range(nchunk):
            r0 = base + c * ch_rows
            pltpu.sync_copy(x_hbm.at[pl.ds(r0, ch_rows)], idx_v)
            pltpu.sync_copy(table_hbm.at[idx_v], rows_v)

            def add_pos(r, _):
                for col in range(D // 16):
                    p = pos_v[r, pl.ds(col * 16, 16)]
                    for k in range(CH):
                        plsc.addupdate(
                            rows_v.at[k * S + r, pl.ds(col * 16, 16)], p)
                return _

            lax.fori_loop(0, S, add_pos, 0)
            pltpu.sync_copy(rows_v, out_hbm.at[pl.ds(r0, ch_rows)])

    return emb


def kernel(x, text_table, pos_table):
    B, S = x.shape
    V, D = text_table.shape
    xf = x.reshape(B * S).astype(jnp.int32)
    emb = _build(B, S, V, D)
    out = emb(xf, text_table, pos_table)
    return out.reshape(B, S, D)


# speedup vs baseline: 4.1841x; 1.1359x over previous
import functools

import jax
import jax.numpy as jnp
from jax import lax
from jax.experimental import pallas as pl
from jax.experimental.pallas import tpu as pltpu
from jax.experimental.pallas import tpu_sc as plsc

NC = 2
NS = 16
NW = NC * NS

CH = 2
NBUF = 4


def _build(B, S, V, D):
    rows = B * S
    assert rows % NW == 0
    rows_per_w = rows // NW
    ch_rows = CH * S
    assert rows_per_w % ch_rows == 0
    nchunk = rows_per_w // ch_rows

    mesh = plsc.VectorSubcoreMesh(core_axis_name="c", subcore_axis_name="s")

    @functools.partial(
        pl.kernel,
        mesh=mesh,
        compiler_params=pltpu.CompilerParams(use_tc_tiling_on_sc=False),
        out_type=jax.ShapeDtypeStruct((rows, D), jnp.float32),
        scratch_types=[
            [pltpu.VMEM((ch_rows,), jnp.int32) for _ in range(NBUF)],
            [pltpu.VMEM((ch_rows, D), jnp.float32) for _ in range(NBUF)],
            pltpu.VMEM((S, D), jnp.float32),
            [pltpu.SemaphoreType.DMA for _ in range(NBUF)],
            [pltpu.SemaphoreType.DMA for _ in range(NBUF)],
            [pltpu.SemaphoreType.DMA for _ in range(NBUF)],
        ],
    )
    def emb(x_hbm, table_hbm, pos_hbm, out_hbm,
            idx_v, rows_v, pos_v, sem_i, sem_g, sem_o):
        cid = lax.axis_index("c")
        sid = lax.axis_index("s")
        wid = sid * NC + cid

        pltpu.sync_copy(pos_hbm, pos_v)

        base = wid * rows_per_w

        def add_pos_chunk(p):
            def add_pos(r, _):
                for col in range(D // 16):
                    pv = pos_v[r, pl.ds(col * 16, 16)]
                    for k in range(CH):
                        plsc.addupdate(
                            rows_v[p].at[k * S + r, pl.ds(col * 16, 16)], pv)
                return _
            lax.fori_loop(0, S, add_pos, 0)

        d_i, d_g, d_o = {}, {}, {}

        def issue_idx(c):
            p = c % NBUF
            d_i[c] = pltpu.async_copy(
                x_hbm.at[pl.ds(base + c * ch_rows, ch_rows)],
                idx_v[p], sem_i[p])

        def issue_gather(c):
            p = c % NBUF
            d_i[c].wait()
            if c - NBUF >= 0:
                d_o[c - NBUF].wait()
            d_g[c] = pltpu.async_copy(
                table_hbm.at[idx_v[p]], rows_v[p], sem_g[p])

        for c in range(min(NBUF, nchunk)):
            issue_idx(c)
        issue_gather(0)

        for c in range(nchunk):
            p = c % NBUF
            d_g[c].wait()
            if c + 1 < nchunk:
                issue_gather(c + 1)
            if c + NBUF < nchunk:
                issue_idx(c + NBUF)
            add_pos_chunk(p)
            d_o[c] = pltpu.async_copy(
                rows_v[p], out_hbm.at[pl.ds(base + c * ch_rows, ch_rows)],
                sem_o[p])

        for c in range(max(0, nchunk - NBUF), nchunk):
            d_o[c].wait()

    return emb


def kernel(x, text_table, pos_table):
    B, S = x.shape
    V, D = text_table.shape
    xf = x.reshape(B * S).astype(jnp.int32)
    emb = _build(B, S, V, D)
    out = emb(xf, text_table, pos_table)
    return out.reshape(B, S, D)


# speedup vs baseline: 4.1893x; 1.0012x over previous
import functools

import jax
import jax.numpy as jnp
from jax import lax
from jax.experimental import pallas as pl
from jax.experimental.pallas import tpu as pltpu
from jax.experimental.pallas import tpu_sc as plsc

NC = 2
NS = 16
NW = NC * NS

CH = 2
NBUF = 4


def _build(B, S, V, D):
    rows = B * S
    assert rows % NW == 0
    rows_per_w = rows // NW
    ch_rows = CH * S
    assert rows_per_w % ch_rows == 0
    nchunk = rows_per_w // ch_rows

    mesh = plsc.VectorSubcoreMesh(core_axis_name="c", subcore_axis_name="s")

    @functools.partial(
        pl.kernel,
        mesh=mesh,
        compiler_params=pltpu.CompilerParams(use_tc_tiling_on_sc=False),
        out_type=jax.ShapeDtypeStruct((rows, D), jnp.float32),
        scratch_types=[
            [pltpu.VMEM((ch_rows,), jnp.int32) for _ in range(NBUF)],
            [pltpu.VMEM((ch_rows, D), jnp.float32) for _ in range(NBUF)],
            pltpu.VMEM((S, D), jnp.float32),
            [pltpu.SemaphoreType.DMA for _ in range(NBUF)],
            [pltpu.SemaphoreType.DMA for _ in range(NBUF)],
            [pltpu.SemaphoreType.DMA for _ in range(NBUF)],
        ],
    )
    def emb(x_hbm, table_hbm, pos_hbm, out_hbm,
            idx_v, rows_v, pos_v, sem_i, sem_g, sem_o):
        cid = lax.axis_index("c")
        sid = lax.axis_index("s")
        wid = sid * NC + cid

        pltpu.sync_copy(pos_hbm, pos_v)

        base = wid * rows_per_w

        def add_pos_chunk(p):
            @plsc.parallel_loop(0, S, unroll=4)
            def add_pos(r):
                for col in range(D // 16):
                    pv = pos_v[r, pl.ds(col * 16, 16)]
                    for k in range(CH):
                        plsc.addupdate(
                            rows_v[p].at[k * S + r, pl.ds(col * 16, 16)], pv)

        d_i, d_g, d_o = {}, {}, {}

        def issue_idx(c):
            p = c % NBUF
            d_i[c] = pltpu.async_copy(
                x_hbm.at[pl.ds(base + c * ch_rows, ch_rows)],
                idx_v[p], sem_i[p])

        def issue_gather(c):
            p = c % NBUF
            d_i[c].wait()
            if c - NBUF >= 0:
                d_o[c - NBUF].wait()
            d_g[c] = pltpu.async_copy(
                table_hbm.at[idx_v[p]], rows_v[p], sem_g[p])

        G = 2
        for c in range(min(NBUF, nchunk)):
            issue_idx(c)
        for c in range(min(G, nchunk)):
            issue_gather(c)

        for c in range(nchunk):
            p = c % NBUF
            d_g[c].wait()
            if c + G < nchunk:
                issue_gather(c + G)
            if c + NBUF < nchunk:
                issue_idx(c + NBUF)
            add_pos_chunk(p)
            d_o[c] = pltpu.async_copy(
                rows_v[p], out_hbm.at[pl.ds(base + c * ch_rows, ch_rows)],
                sem_o[p])

        for c in range(max(0, nchunk - NBUF), nchunk):
            d_o[c].wait()

    return emb


def kernel(x, text_table, pos_table):
    B, S = x.shape
    V, D = text_table.shape
    xf = x.reshape(B * S).astype(jnp.int32)
    emb = _build(B, S, V, D)
    out = emb(xf, text_table, pos_table)
    return out.reshape(B, S, D)
